# baseline (speedup 1.0000x reference)
import functools

import jax
import jax.numpy as jnp
from jax import lax
from jax.experimental import pallas as pl
from jax.experimental.pallas import tpu as pltpu
from jax.experimental.pallas import tpu_sc as plsc

N = 10000
E = 320000
D = 128
NC = 2
NS = 16
C = 80
NG = 5
GRP = 25
NPAD = 10240
SLC = NPAD // NS
ZROWS = 624
TAIL = N - NS * ZROWS


def _zero_vec16():
    return jnp.zeros((16,), jnp.float32)


def _scale_rows(gbuf, b, w_ref, j):
    def grp(g, carry):
        wvec = w_ref[j, pl.ds(pl.multiple_of(g * 16, 16), 16)]
        for l in range(16):
            ws = wvec[l]
            e2 = g * 16 + l
            for q in range(D // 16):
                sl = pl.ds(q * 16, 16)
                gbuf[b, e2, sl] = gbuf[b, e2, sl] * ws
        return carry
    lax.fori_loop(0, C // 16, grp, 0)


_sc_mesh = plsc.VectorSubcoreMesh(core_axis_name="c", subcore_axis_name="s")


@functools.partial(
    pl.kernel,
    out_type=[
        jax.ShapeDtypeStruct((NC, NPAD), jnp.float32),
        jax.ShapeDtypeStruct((NC, NS, NG, GRP, C), jnp.float32),
    ],
    mesh=_sc_mesh,
    scratch_types=[
        pltpu.VMEM((NG, GRP, C), jnp.int32),
        pltpu.VMEM((NG, GRP, C), jnp.int32),
        pltpu.VMEM((NG, GRP, C), jnp.float32),
        pltpu.VMEM((SLC,), jnp.float32),
        pltpu.VMEM_SHARED((NPAD,), jnp.float32),
        pltpu.SemaphoreType.DMA,
    ],
)
def _sc_degree(row_h, col_h, ew_h, deg_out, mew_out,
               row_v, col_v, ewm_v, slice_v, deg_s, sem):
    c = lax.axis_index("c")
    s = lax.axis_index("s")

    pltpu.sync_copy(row_h.at[c, s], row_v)
    pltpu.sync_copy(col_h.at[c, s], col_v)
    pltpu.sync_copy(ew_h.at[c, s], ewm_v)

    for k in range(SLC // 16):
        slice_v[pl.ds(k * 16, 16)] = _zero_vec16()
    pltpu.sync_copy(slice_v, deg_s.at[pl.ds(s * SLC, SLC)])

    def mask_g(g, carry):
        def mask_body(j, carry2):
            for k in range(C // 16):
                sl = pl.ds(k * 16, 16)
                ewm_v[g, j, sl] = jnp.where(row_v[g, j, sl] == col_v[g, j, sl],
                                            0.0, ewm_v[g, j, sl])
            return carry2
        return lax.fori_loop(0, GRP, mask_body, carry)
    lax.fori_loop(0, NG, mask_g, 0)

    plsc.subcore_barrier()

    def deg_g(g, carry):
        def deg_body(j, carry2):
            pltpu.sync_copy(ewm_v.at[g, j], deg_s.at[row_v.at[g, j]],
                            add=True)
            return carry2
        return lax.fori_loop(0, GRP, deg_body, carry)
    lax.fori_loop(0, NG, deg_g, 0)

    plsc.subcore_barrier()

    pltpu.sync_copy(deg_s.at[pl.ds(s * SLC, SLC)],
                    deg_out.at[c, pl.ds(s * SLC, SLC)])
    pltpu.sync_copy(ewm_v, mew_out.at[c, s])


@functools.partial(
    pl.kernel,
    out_type=jax.ShapeDtypeStruct((NC, N, D), jnp.float32),
    mesh=_sc_mesh,
    scratch_types=[
        pltpu.VMEM((GRP, C), jnp.int32),
        pltpu.VMEM((GRP, C), jnp.int32),
        pltpu.VMEM((GRP, C), jnp.float32),
        pltpu.VMEM((2, C, D), jnp.float32),
        pltpu.VMEM_SHARED((N, D), jnp.float32),
        pltpu.SemaphoreType.DMA((2,)),
        pltpu.SemaphoreType.DMA((2,)),
    ],
)
def _sc_prop(row_h, col_h, w_h, h_h, u_out,
             row_v, col_v, w_v, gbuf, acc_s, sem_g, sem_s):
    c = lax.axis_index("c")
    s = lax.axis_index("s")

    def zrow(i, carry):
        for q in range(D // 16):
            gbuf[0, i, pl.ds(q * 16, 16)] = _zero_vec16()
        return carry
    lax.fori_loop(0, C, zrow, 0)
    base = s * ZROWS
    for t in range(ZROWS // C):
        pltpu.sync_copy(gbuf.at[0], acc_s.at[pl.ds(base + t * C, C)])
    rem = ZROWS % C
    pltpu.sync_copy(gbuf.at[0, pl.ds(0, rem)],
                    acc_s.at[pl.ds(base + (ZROWS // C) * C, rem)])

    @pl.when(s == NS - 1)
    def _():
        pltpu.sync_copy(gbuf.at[0, pl.ds(0, TAIL)],
                        acc_s.at[pl.ds(NS * ZROWS, TAIL)])

    plsc.subcore_barrier()

    def _wait_sct1():
        pltpu.make_async_copy(gbuf.at[1], acc_s.at[col_v.at[GRP - 2]],
                              sem_s.at[1]).wait()

    def group_body(g, carry):
        @pl.when(g > 0)
        def _():
            _wait_sct1()
        pltpu.sync_copy(row_h.at[c, s, g], row_v)
        pltpu.sync_copy(col_h.at[c, s, g], col_v)
        pltpu.sync_copy(w_h.at[c, s, g], w_v)
        pltpu.async_copy(h_h.at[row_v.at[0]], gbuf.at[0], sem_g.at[0])

        def pair_body(i, carry2):
            j0 = 2 * i
            j1 = j0 + 1
            pltpu.make_async_copy(h_h.at[row_v.at[j0]], gbuf.at[0],
                                  sem_g.at[0]).wait()

            @pl.when(i > 0)
            def _():
                _wait_sct1()
            pltpu.async_copy(h_h.at[row_v.at[j1]], gbuf.at[1], sem_g.at[1])
            _scale_rows(gbuf, 0, w_v, j0)
            pltpu.async_copy(gbuf.at[0], acc_s.at[col_v.at[j0]],
                             sem_s.at[0], add=True)
            pltpu.make_async_copy(h_h.at[row_v.at[j1]], gbuf.at[1],
                                  sem_g.at[1]).wait()
            _scale_rows(gbuf, 1, w_v, j1)
            pltpu.make_async_copy(gbuf.at[0], acc_s.at[col_v.at[j0]],
                                  sem_s.at[0]).wait()
            pltpu.async_copy(h_h.at[row_v.at[j0 + 2]], gbuf.at[0],
                             sem_g.at[0])
            pltpu.async_copy(gbuf.at[1], acc_s.at[col_v.at[j1]],
                             sem_s.at[1], add=True)
            return carry2
        lax.fori_loop(0, (GRP - 1) // 2, pair_body, carry)
        pltpu.make_async_copy(h_h.at[row_v.at[GRP - 1]], gbuf.at[0],
                              sem_g.at[0]).wait()
        _scale_rows(gbuf, 0, w_v, GRP - 1)
        pltpu.sync_copy(gbuf.at[0], acc_s.at[col_v.at[GRP - 1]], add=True)
        return carry
    lax.fori_loop(0, NG, group_body, 0)
    _wait_sct1()

    plsc.subcore_barrier()

    pltpu.sync_copy(acc_s.at[pl.ds(base, ZROWS)],
                    u_out.at[c, pl.ds(base, ZROWS)])

    @pl.when(s == NS - 1)
    def _():
        pltpu.sync_copy(acc_s.at[pl.ds(NS * ZROWS, TAIL)],
                        u_out.at[c, pl.ds(NS * ZROWS, TAIL)])


_BLK = 2000


def _tc_pre_body(x, d0, d1, w0, wlt, bl, y_o, dinv_o, xw0_o, xl_o):
    deg = d0[...] + d1[...]
    dinv = jnp.where(deg > 0, lax.rsqrt(jnp.where(deg > 0, deg, 1.0)), 0.0)
    dinv_o[...] = dinv
    xb = x[...]
    y_o[...] = xb * dinv
    xw0_o[...] = jnp.dot(xb, w0[...], preferred_element_type=jnp.float32)
    xl_o[...] = jnp.dot(xb, wlt[...], preferred_element_type=jnp.float32) + bl[...]


def _tc_mid_body(u0, u1, dinv, xw0, w1, s1_o, y2_o):
    dv = dinv[...]
    tx1 = -(u0[...] + u1[...]) * dv
    s1_o[...] = xw0[...] + jnp.dot(tx1, w1[...],
                                   preferred_element_type=jnp.float32)
    y2_o[...] = tx1 * dv


def _tc_final_body(u0, u1, dinv, x, s1, xl, w2, bc, out_o):
    tx2 = -2.0 * (u0[...] + u1[...]) * dinv[...] - x[...]
    g = (s1[...] + jnp.dot(tx2, w2[...], preferred_element_type=jnp.float32)
         + bc[...])
    out_o[...] = jnp.where(g >= 0, g, 0.01 * g) + xl[...]


def _row_spec():
    return pl.BlockSpec((_BLK, D), lambda i: (i, 0))


def _col_spec():
    return pl.BlockSpec((_BLK, 1), lambda i: (i, 0))


def _w_spec():
    return pl.BlockSpec((D, D), lambda i: (0, 0))


def _b_spec():
    return pl.BlockSpec((1, D), lambda i: (0, 0))


def kernel(x, edge_index, edge_attr, W_cheb, b_cheb, W_lin, b_lin):
    row4 = edge_index[0].reshape(NC, NS, NG, GRP, C)
    col4 = edge_index[1].reshape(NC, NS, NG, GRP, C)
    ew4 = edge_attr.reshape(NC, NS, NG, GRP, C)

    degp, mew4 = _sc_degree(row4, col4, ew4)
    d0 = degp[0, :N].reshape(N, 1)
    d1 = degp[1, :N].reshape(N, 1)

    grid = (N // _BLK,)
    y, dinv, xw0, xl = pl.pallas_call(
        _tc_pre_body,
        grid=grid,
        in_specs=[_row_spec(), _col_spec(), _col_spec(),
                  _w_spec(), _w_spec(), _b_spec()],
        out_specs=[_row_spec(), _col_spec(), _row_spec(), _row_spec()],
        out_shape=[
            jax.ShapeDtypeStruct((N, D), jnp.float32),
            jax.ShapeDtypeStruct((N, 1), jnp.float32),
            jax.ShapeDtypeStruct((N, D), jnp.float32),
            jax.ShapeDtypeStruct((N, D), jnp.float32),
        ],
    )(x, d0, d1, W_cheb[0], W_lin.T, b_lin.reshape(1, D))

    u1 = _sc_prop(row4, col4, mew4, y)

    s1, y2 = pl.pallas_call(
        _tc_mid_body,
        grid=grid,
        in_specs=[_row_spec(), _row_spec(), _col_spec(),
                  _row_spec(), _w_spec()],
        out_specs=[_row_spec(), _row_spec()],
        out_shape=[jax.ShapeDtypeStruct((N, D), jnp.float32)] * 2,
    )(u1[0], u1[1], dinv, xw0, W_cheb[1])

    u2 = _sc_prop(row4, col4, mew4, y2)

    out = pl.pallas_call(
        _tc_final_body,
        grid=grid,
        in_specs=[_row_spec(), _row_spec(), _col_spec(), _row_spec(),
                  _row_spec(), _row_spec(), _w_spec(), _b_spec()],
        out_specs=_row_spec(),
        out_shape=jax.ShapeDtypeStruct((N, D), jnp.float32),
    )(u2[0], u2[1], dinv, x, s1, xl, W_cheb[2], b_cheb.reshape(1, D))
    return out


# speedup vs baseline: 1.0042x; 1.0042x over previous
import functools

import jax
import jax.numpy as jnp
from jax import lax
from jax.experimental import pallas as pl
from jax.experimental.pallas import tpu as pltpu
from jax.experimental.pallas import tpu_sc as plsc

N = 10000
E = 320000
D = 128
NC = 2
NS = 16
C = 80
NG = 5
GRP = 25
NPAD = 10240
SLC = NPAD // NS
ZROWS = 624
TAIL = N - NS * ZROWS


def _zero_vec16():
    return jnp.zeros((16,), jnp.float32)


def _scale_rows(gbuf, b, w_ref, j):
    def grp(g, carry):
        wvec = w_ref[j, pl.ds(pl.multiple_of(g * 16, 16), 16)]
        for l in range(16):
            ws = wvec[l]
            e2 = g * 16 + l
            for q in range(D // 16):
                sl = pl.ds(q * 16, 16)
                gbuf[b, e2, sl] = gbuf[b, e2, sl] * ws
        return carry
    lax.fori_loop(0, C // 16, grp, 0)


_sc_mesh = plsc.VectorSubcoreMesh(core_axis_name="c", subcore_axis_name="s")


@functools.partial(
    pl.kernel,
    out_type=[
        jax.ShapeDtypeStruct((NC, NPAD), jnp.float32),
        jax.ShapeDtypeStruct((NC, NS, NG, GRP, C), jnp.float32),
    ],
    mesh=_sc_mesh,
    scratch_types=[
        pltpu.VMEM((NG, GRP, C), jnp.int32),
        pltpu.VMEM((NG, GRP, C), jnp.int32),
        pltpu.VMEM((NG, GRP, C), jnp.float32),
        pltpu.VMEM((SLC,), jnp.float32),
        pltpu.VMEM_SHARED((NPAD,), jnp.float32),
        pltpu.SemaphoreType.DMA,
    ],
)
def _sc_degree(row_h, col_h, ew_h, deg_out, mew_out,
               row_v, col_v, ewm_v, slice_v, deg_s, sem):
    c = lax.axis_index("c")
    s = lax.axis_index("s")

    pltpu.sync_copy(row_h.at[c, s], row_v)
    pltpu.sync_copy(col_h.at[c, s], col_v)
    pltpu.sync_copy(ew_h.at[c, s], ewm_v)

    for k in range(SLC // 16):
        slice_v[pl.ds(k * 16, 16)] = _zero_vec16()
    pltpu.sync_copy(slice_v, deg_s.at[pl.ds(s * SLC, SLC)])

    def mask_g(g, carry):
        def mask_body(j, carry2):
            for k in range(C // 16):
                sl = pl.ds(k * 16, 16)
                ewm_v[g, j, sl] = jnp.where(row_v[g, j, sl] == col_v[g, j, sl],
                                            0.0, ewm_v[g, j, sl])
            return carry2
        return lax.fori_loop(0, GRP, mask_body, carry)
    lax.fori_loop(0, NG, mask_g, 0)

    plsc.subcore_barrier()

    def deg_g(g, carry):
        def deg_body(j, carry2):
            pltpu.sync_copy(ewm_v.at[g, j], deg_s.at[row_v.at[g, j]],
                            add=True)
            return carry2
        return lax.fori_loop(0, GRP, deg_body, carry)
    lax.fori_loop(0, NG, deg_g, 0)

    plsc.subcore_barrier()

    pltpu.sync_copy(deg_s.at[pl.ds(s * SLC, SLC)],
                    deg_out.at[c, pl.ds(s * SLC, SLC)])
    pltpu.sync_copy(ewm_v, mew_out.at[c, s])


@functools.partial(
    pl.kernel,
    out_type=jax.ShapeDtypeStruct((NC, N, D), jnp.float32),
    mesh=_sc_mesh,
    scratch_types=[
        pltpu.VMEM((GRP, C), jnp.int32),
        pltpu.VMEM((GRP, C), jnp.int32),
        pltpu.VMEM((GRP, C), jnp.float32),
        pltpu.VMEM((2, C, D), jnp.float32),
        pltpu.VMEM_SHARED((N, D), jnp.float32),
        pltpu.SemaphoreType.DMA((2,)),
        pltpu.SemaphoreType.DMA,
    ],
)
def _sc_prop(row_h, col_h, w_h, h_h, u_out,
             row_v, col_v, w_v, gbuf, acc_s, sem_g, sem_s):
    c = lax.axis_index("c")
    s = lax.axis_index("s")

    def zrow(i, carry):
        for q in range(D // 16):
            gbuf[0, i, pl.ds(q * 16, 16)] = _zero_vec16()
        return carry
    lax.fori_loop(0, C, zrow, 0)
    base = s * ZROWS
    for t in range(ZROWS // C):
        pltpu.sync_copy(gbuf.at[0], acc_s.at[pl.ds(base + t * C, C)])
    rem = ZROWS % C
    pltpu.sync_copy(gbuf.at[0, pl.ds(0, rem)],
                    acc_s.at[pl.ds(base + (ZROWS // C) * C, rem)])

    @pl.when(s == NS - 1)
    def _():
        pltpu.sync_copy(gbuf.at[0, pl.ds(0, TAIL)],
                        acc_s.at[pl.ds(NS * ZROWS, TAIL)])

    plsc.subcore_barrier()

    def group_body(g, carry):
        pltpu.sync_copy(row_h.at[c, s, g], row_v)
        pltpu.sync_copy(col_h.at[c, s, g], col_v)
        pltpu.sync_copy(w_h.at[c, s, g], w_v)
        pltpu.async_copy(h_h.at[row_v.at[0]], gbuf.at[0], sem_g.at[0])

        def pair_body(i, carry2):
            j0 = 2 * i
            j1 = j0 + 1
            pltpu.make_async_copy(h_h.at[row_v.at[j0]], gbuf.at[0],
                                  sem_g.at[0]).wait()
            pltpu.async_copy(h_h.at[row_v.at[j1]], gbuf.at[1], sem_g.at[1])
            _scale_rows(gbuf, 0, w_v, j0)
            pltpu.async_copy(gbuf.at[0], acc_s.at[col_v.at[j0]], sem_s,
                             add=True)
            pltpu.make_async_copy(h_h.at[row_v.at[j1]], gbuf.at[1],
                                  sem_g.at[1]).wait()
            _scale_rows(gbuf, 1, w_v, j1)
            pltpu.make_async_copy(gbuf.at[0], acc_s.at[col_v.at[j0]],
                                  sem_s).wait()
            pltpu.async_copy(h_h.at[row_v.at[j0 + 2]], gbuf.at[0],
                             sem_g.at[0])
            pltpu.sync_copy(gbuf.at[1], acc_s.at[col_v.at[j1]], add=True)
            return carry2
        lax.fori_loop(0, (GRP - 1) // 2, pair_body, carry)
        pltpu.make_async_copy(h_h.at[row_v.at[GRP - 1]], gbuf.at[0],
                              sem_g.at[0]).wait()
        _scale_rows(gbuf, 0, w_v, GRP - 1)
        pltpu.sync_copy(gbuf.at[0], acc_s.at[col_v.at[GRP - 1]], add=True)
        return carry
    lax.fori_loop(0, NG, group_body, 0)

    plsc.subcore_barrier()

    pltpu.sync_copy(acc_s.at[pl.ds(base, ZROWS)],
                    u_out.at[c, pl.ds(base, ZROWS)])

    @pl.when(s == NS - 1)
    def _():
        pltpu.sync_copy(acc_s.at[pl.ds(NS * ZROWS, TAIL)],
                        u_out.at[c, pl.ds(NS * ZROWS, TAIL)])


_BLK = 2000


def _tc_pre_body(x, d0, d1, w0, wlt, bl, y_o, dinv_o, xw0_o, xl_o):
    deg = d0[...] + d1[...]
    dinv = jnp.where(deg > 0, lax.rsqrt(jnp.where(deg > 0, deg, 1.0)), 0.0)
    dinv_o[...] = dinv
    xb = x[...]
    y_o[...] = xb * dinv
    xw0_o[...] = jnp.dot(xb, w0[...], preferred_element_type=jnp.float32)
    xl_o[...] = jnp.dot(xb, wlt[...], preferred_element_type=jnp.float32) + bl[...]


def _tc_mid_body(u0, u1, dinv, xw0, w1, s1_o, y2_o):
    dv = dinv[...]
    tx1 = -(u0[...] + u1[...]) * dv
    s1_o[...] = xw0[...] + jnp.dot(tx1, w1[...],
                                   preferred_element_type=jnp.float32)
    y2_o[...] = tx1 * dv


def _tc_final_body(u0, u1, dinv, x, s1, xl, w2, bc, out_o):
    tx2 = -2.0 * (u0[...] + u1[...]) * dinv[...] - x[...]
    g = (s1[...] + jnp.dot(tx2, w2[...], preferred_element_type=jnp.float32)
         + bc[...])
    out_o[...] = jnp.where(g >= 0, g, 0.01 * g) + xl[...]


def _row_spec():
    return pl.BlockSpec((_BLK, D), lambda i: (i, 0))


def _col_spec():
    return pl.BlockSpec((_BLK, 1), lambda i: (i, 0))


def _w_spec():
    return pl.BlockSpec((D, D), lambda i: (0, 0))


def _b_spec():
    return pl.BlockSpec((1, D), lambda i: (0, 0))


def kernel(x, edge_index, edge_attr, W_cheb, b_cheb, W_lin, b_lin):
    row4 = edge_index[0].reshape(NC, NS, NG, GRP, C)
    col4 = edge_index[1].reshape(NC, NS, NG, GRP, C)
    ew4 = edge_attr.reshape(NC, NS, NG, GRP, C)

    degp, mew4 = _sc_degree(row4, col4, ew4)
    d0 = degp[0, :N].reshape(N, 1)
    d1 = degp[1, :N].reshape(N, 1)

    grid = (N // _BLK,)
    y, dinv, xw0, xl = pl.pallas_call(
        _tc_pre_body,
        grid=grid,
        in_specs=[_row_spec(), _col_spec(), _col_spec(),
                  _w_spec(), _w_spec(), _b_spec()],
        out_specs=[_row_spec(), _col_spec(), _row_spec(), _row_spec()],
        out_shape=[
            jax.ShapeDtypeStruct((N, D), jnp.float32),
            jax.ShapeDtypeStruct((N, 1), jnp.float32),
            jax.ShapeDtypeStruct((N, D), jnp.float32),
            jax.ShapeDtypeStruct((N, D), jnp.float32),
        ],
    )(x, d0, d1, W_cheb[0], W_lin.T, b_lin.reshape(1, D))

    u1 = _sc_prop(row4, col4, mew4, y)

    s1, y2 = pl.pallas_call(
        _tc_mid_body,
        grid=grid,
        in_specs=[_row_spec(), _row_spec(), _col_spec(),
                  _row_spec(), _w_spec()],
        out_specs=[_row_spec(), _row_spec()],
        out_shape=[jax.ShapeDtypeStruct((N, D), jnp.float32)] * 2,
    )(u1[0], u1[1], dinv, xw0, W_cheb[1])

    u2 = _sc_prop(row4, col4, mew4, y2)

    out = pl.pallas_call(
        _tc_final_body,
        grid=grid,
        in_specs=[_row_spec(), _row_spec(), _col_spec(), _row_spec(),
                  _row_spec(), _row_spec(), _w_spec(), _b_spec()],
        out_specs=_row_spec(),
        out_shape=jax.ShapeDtypeStruct((N, D), jnp.float32),
    )(u2[0], u2[1], dinv, x, s1, xl, W_cheb[2], b_cheb.reshape(1, D))
    return out


# speedup vs baseline: 1.2161x; 1.2110x over previous
import functools

import jax
import jax.numpy as jnp
from jax import lax
from jax.experimental import pallas as pl
from jax.experimental.pallas import tpu as pltpu
from jax.experimental.pallas import tpu_sc as plsc

N = 10000
E = 320000
D = 128
NC = 2
NS = 16
C = 80
NG = 5
GRP = 25
EW = E // (NC * NS)
GW = GRP * C
NPAD = 10240
SLC = NPAD // NS
ZROWS = 624
TAIL = N - NS * ZROWS


def _zero_vec16():
    return jnp.zeros((16,), jnp.float32)


def _scale_rows(gbuf, b, w_ref, j):
    def grp(g, carry):
        wvec = w_ref[pl.ds(j * C + g * 16, 16)]
        for l in range(16):
            ws = wvec[l]
            e2 = g * 16 + l
            for q in range(D // 16):
                sl = pl.ds(q * 16, 16)
                gbuf[b, e2, sl] = gbuf[b, e2, sl] * ws
        return carry
    lax.fori_loop(0, C // 16, grp, 0)


def _relayout_idx(src1d, dst2d, nrows):
    def body(j, carry):
        for k in range(C // 16):
            dst2d[j, pl.ds(k * 16, 16)] = src1d[pl.ds(j * C + k * 16, 16)]
        return carry
    lax.fori_loop(0, nrows, body, 0)


_sc_mesh = plsc.VectorSubcoreMesh(core_axis_name="c", subcore_axis_name="s")


@functools.partial(
    pl.kernel,
    out_type=[
        jax.ShapeDtypeStruct((NC, NPAD), jnp.float32),
        jax.ShapeDtypeStruct((E,), jnp.float32),
    ],
    mesh=_sc_mesh,
    scratch_types=[
        pltpu.VMEM((EW,), jnp.int32),
        pltpu.VMEM((EW,), jnp.int32),
        pltpu.VMEM((EW,), jnp.float32),
        pltpu.VMEM((EW // C, C), jnp.int32),
        pltpu.VMEM((SLC,), jnp.float32),
        pltpu.VMEM_SHARED((NPAD,), jnp.float32),
        pltpu.SemaphoreType.DMA,
    ],
)
def _sc_degree(row_h, col_h, ew_h, deg_out, mew_out,
               row_v, col_v, ewm_v, row2_v, slice_v, deg_s, sem):
    c = lax.axis_index("c")
    s = lax.axis_index("s")
    off = (c * NS + s) * EW

    pltpu.sync_copy(row_h.at[pl.ds(off, EW)], row_v)
    pltpu.sync_copy(col_h.at[pl.ds(off, EW)], col_v)
    pltpu.sync_copy(ew_h.at[pl.ds(off, EW)], ewm_v)

    for k in range(SLC // 16):
        slice_v[pl.ds(k * 16, 16)] = _zero_vec16()
    pltpu.sync_copy(slice_v, deg_s.at[pl.ds(s * SLC, SLC)])

    def mask_body(i, carry):
        sl = pl.ds(i * 16, 16)
        ewm_v[sl] = jnp.where(row_v[sl] == col_v[sl], 0.0, ewm_v[sl])
        return carry
    lax.fori_loop(0, EW // 16, mask_body, 0)
    _relayout_idx(row_v, row2_v, EW // C)

    plsc.subcore_barrier()

    def deg_body(j, carry):
        pltpu.sync_copy(ewm_v.at[pl.ds(j * C, C)], deg_s.at[row2_v.at[j]],
                        add=True)
        return carry
    lax.fori_loop(0, EW // C, deg_body, 0)

    plsc.subcore_barrier()

    pltpu.sync_copy(deg_s.at[pl.ds(s * SLC, SLC)],
                    deg_out.at[c, pl.ds(s * SLC, SLC)])
    pltpu.sync_copy(ewm_v, mew_out.at[pl.ds(off, EW)])


@functools.partial(
    pl.kernel,
    out_type=jax.ShapeDtypeStruct((NC, N, D), jnp.float32),
    mesh=_sc_mesh,
    scratch_types=[
        pltpu.VMEM((GW,), jnp.int32),
        pltpu.VMEM((GW,), jnp.int32),
        pltpu.VMEM((GRP, C), jnp.int32),
        pltpu.VMEM((GW,), jnp.float32),
        pltpu.VMEM((2, C, D), jnp.float32),
        pltpu.VMEM_SHARED((N, D), jnp.float32),
        pltpu.SemaphoreType.DMA((2,)),
    ],
)
def _sc_prop(row_h, col_h, w_h, h_h, u_out,
             row_v, col_v, col2_v, w_v, gbuf, acc_s, sem_g):
    c = lax.axis_index("c")
    s = lax.axis_index("s")
    off = (c * NS + s) * EW

    def zrow(i, carry):
        for q in range(D // 16):
            gbuf[0, i, pl.ds(q * 16, 16)] = _zero_vec16()
        return carry
    lax.fori_loop(0, C, zrow, 0)
    base = s * ZROWS
    for t in range(ZROWS // C):
        pltpu.sync_copy(gbuf.at[0], acc_s.at[pl.ds(base + t * C, C)])
    rem = ZROWS % C
    pltpu.sync_copy(gbuf.at[0, pl.ds(0, rem)],
                    acc_s.at[pl.ds(base + (ZROWS // C) * C, rem)])

    @pl.when(s == NS - 1)
    def _():
        pltpu.sync_copy(gbuf.at[0, pl.ds(0, TAIL)],
                        acc_s.at[pl.ds(NS * ZROWS, TAIL)])

    plsc.subcore_barrier()

    def _ridx(j):
        return row_v.at[pl.ds(j * C, C)]

    def group_body(g, carry):
        goff = off + g * GW
        pltpu.sync_copy(row_h.at[pl.ds(goff, GW)], row_v)
        pltpu.sync_copy(col_h.at[pl.ds(goff, GW)], col_v)
        pltpu.sync_copy(w_h.at[pl.ds(goff, GW)], w_v)
        _relayout_idx(col_v, col2_v, GRP)
        pltpu.async_copy(h_h.at[_ridx(0)], gbuf.at[0], sem_g.at[0])

        def pair_body(i, carry2):
            j0 = 2 * i
            j1 = j0 + 1
            pltpu.make_async_copy(h_h.at[_ridx(j0)], gbuf.at[0],
                                  sem_g.at[0]).wait()
            pltpu.async_copy(h_h.at[_ridx(j1)], gbuf.at[1], sem_g.at[1])
            _scale_rows(gbuf, 0, w_v, j0)
            pltpu.sync_copy(gbuf.at[0], acc_s.at[col2_v.at[j0]], add=True)
            pltpu.async_copy(h_h.at[_ridx(j0 + 2)], gbuf.at[0],
                             sem_g.at[0])
            pltpu.make_async_copy(h_h.at[_ridx(j1)], gbuf.at[1],
                                  sem_g.at[1]).wait()
            _scale_rows(gbuf, 1, w_v, j1)
            pltpu.sync_copy(gbuf.at[1], acc_s.at[col2_v.at[j1]], add=True)
            return carry2
        lax.fori_loop(0, (GRP - 1) // 2, pair_body, carry)
        pltpu.make_async_copy(h_h.at[_ridx(GRP - 1)], gbuf.at[0],
                              sem_g.at[0]).wait()
        _scale_rows(gbuf, 0, w_v, GRP - 1)
        pltpu.sync_copy(gbuf.at[0], acc_s.at[col2_v.at[GRP - 1]], add=True)
        return carry
    lax.fori_loop(0, NG, group_body, 0)

    plsc.subcore_barrier()

    pltpu.sync_copy(acc_s.at[pl.ds(base, ZROWS)],
                    u_out.at[c, pl.ds(base, ZROWS)])

    @pl.when(s == NS - 1)
    def _():
        pltpu.sync_copy(acc_s.at[pl.ds(NS * ZROWS, TAIL)],
                        u_out.at[c, pl.ds(NS * ZROWS, TAIL)])


_BLK = 2000


def _tc_pre_body(x, d0, d1, w0, wlt, bl, y_o, dinv_o, xw0_o, xl_o):
    deg = d0[...] + d1[...]
    dinv = jnp.where(deg > 0, lax.rsqrt(jnp.where(deg > 0, deg, 1.0)), 0.0)
    dinv_o[...] = dinv
    xb = x[...]
    y_o[...] = xb * dinv
    xw0_o[...] = jnp.dot(xb, w0[...], preferred_element_type=jnp.float32)
    xl_o[...] = jnp.dot(xb, wlt[...], preferred_element_type=jnp.float32) + bl[...]


def _tc_mid_body(u, dinv, xw0, w1, s1_o, y2_o):
    dv = dinv[...]
    tx1 = -(u[0] + u[1]) * dv
    s1_o[...] = xw0[...] + jnp.dot(tx1, w1[...],
                                   preferred_element_type=jnp.float32)
    y2_o[...] = tx1 * dv


def _tc_final_body(u, dinv, x, s1, xl, w2, bc, out_o):
    tx2 = -2.0 * (u[0] + u[1]) * dinv[...] - x[...]
    g = (s1[...] + jnp.dot(tx2, w2[...], preferred_element_type=jnp.float32)
         + bc[...])
    out_o[...] = jnp.where(g >= 0, g, 0.01 * g) + xl[...]


def _row_spec():
    return pl.BlockSpec((_BLK, D), lambda i: (i, 0))


def _col_spec():
    return pl.BlockSpec((_BLK, 1), lambda i: (i, 0))


def _u_spec():
    return pl.BlockSpec((NC, _BLK, D), lambda i: (0, i, 0))


def _w_spec():
    return pl.BlockSpec((D, D), lambda i: (0, 0))


def _b_spec():
    return pl.BlockSpec((1, D), lambda i: (0, 0))


def kernel(x, edge_index, edge_attr, W_cheb, b_cheb, W_lin, b_lin):
    row1 = edge_index[0]
    col1 = edge_index[1]

    degp, mew = _sc_degree(row1, col1, edge_attr)
    d0 = degp[0, :N].reshape(N, 1)
    d1 = degp[1, :N].reshape(N, 1)

    grid = (N // _BLK,)
    y, dinv, xw0, xl = pl.pallas_call(
        _tc_pre_body,
        grid=grid,
        in_specs=[_row_spec(), _col_spec(), _col_spec(),
                  _w_spec(), _w_spec(), _b_spec()],
        out_specs=[_row_spec(), _col_spec(), _row_spec(), _row_spec()],
        out_shape=[
            jax.ShapeDtypeStruct((N, D), jnp.float32),
            jax.ShapeDtypeStruct((N, 1), jnp.float32),
            jax.ShapeDtypeStruct((N, D), jnp.float32),
            jax.ShapeDtypeStruct((N, D), jnp.float32),
        ],
    )(x, d0, d1, W_cheb[0], W_lin.T, b_lin.reshape(1, D))

    u1 = _sc_prop(row1, col1, mew, y)

    s1, y2 = pl.pallas_call(
        _tc_mid_body,
        grid=grid,
        in_specs=[_u_spec(), _col_spec(), _row_spec(), _w_spec()],
        out_specs=[_row_spec(), _row_spec()],
        out_shape=[jax.ShapeDtypeStruct((N, D), jnp.float32)] * 2,
    )(u1, dinv, xw0, W_cheb[1])

    u2 = _sc_prop(row1, col1, mew, y2)

    out = pl.pallas_call(
        _tc_final_body,
        grid=grid,
        in_specs=[_u_spec(), _col_spec(), _row_spec(),
                  _row_spec(), _row_spec(), _w_spec(), _b_spec()],
        out_specs=_row_spec(),
        out_shape=jax.ShapeDtypeStruct((N, D), jnp.float32),
    )(u2, dinv, x, s1, xl, W_cheb[2], b_cheb.reshape(1, D))
    return out


# speedup vs baseline: 1.2523x; 1.0298x over previous
import functools

import jax
import jax.numpy as jnp
from jax import lax
from jax.experimental import pallas as pl
from jax.experimental.pallas import tpu as pltpu
from jax.experimental.pallas import tpu_sc as plsc

N = 10000
E = 320000
D = 128
NC = 2
NS = 16
C = 80
NG = 5
GRP = 25
EW = E // (NC * NS)
GW = GRP * C
NPAD = 10240
SLC = NPAD // NS
ZROWS = 624
TAIL = N - NS * ZROWS


def _zero_vec16():
    return jnp.zeros((16,), jnp.float32)


def _scale_rows(gbuf, b, w_ref, j):
    def grp(g, carry):
        wvec = w_ref[pl.ds(j * C + g * 16, 16)]
        for l in range(16):
            ws = wvec[l]
            e2 = g * 16 + l
            for q in range(D // 16):
                sl = pl.ds(q * 16, 16)
                gbuf[b, e2, sl] = gbuf[b, e2, sl] * ws
        return carry
    lax.fori_loop(0, C // 16, grp, 0)


def _relayout_idx(src1d, dst2d, nrows):
    def body(j, carry):
        for k in range(C // 16):
            dst2d[j, pl.ds(k * 16, 16)] = src1d[pl.ds(j * C + k * 16, 16)]
        return carry
    lax.fori_loop(0, nrows, body, 0)


_sc_mesh = plsc.VectorSubcoreMesh(core_axis_name="c", subcore_axis_name="s")


@functools.partial(
    pl.kernel,
    out_type=[
        jax.ShapeDtypeStruct((NC, NPAD), jnp.float32),
        jax.ShapeDtypeStruct((E,), jnp.float32),
    ],
    mesh=_sc_mesh,
    scratch_types=[
        pltpu.VMEM((EW,), jnp.int32),
        pltpu.VMEM((EW,), jnp.int32),
        pltpu.VMEM((EW,), jnp.float32),
        pltpu.VMEM((EW // C, C), jnp.int32),
        pltpu.VMEM((SLC,), jnp.float32),
        pltpu.VMEM_SHARED((NPAD,), jnp.float32),
        pltpu.SemaphoreType.DMA,
    ],
)
def _sc_degree(row_h, col_h, ew_h, deg_out, mew_out,
               row_v, col_v, ewm_v, row2_v, slice_v, deg_s, sem):
    c = lax.axis_index("c")
    s = lax.axis_index("s")
    off = (c * NS + s) * EW

    pltpu.async_copy(row_h.at[pl.ds(off, EW)], row_v, sem)
    pltpu.async_copy(col_h.at[pl.ds(off, EW)], col_v, sem)
    pltpu.async_copy(ew_h.at[pl.ds(off, EW)], ewm_v, sem)
    pltpu.make_async_copy(row_h.at[pl.ds(off, EW)], row_v, sem).wait()
    pltpu.make_async_copy(col_h.at[pl.ds(off, EW)], col_v, sem).wait()
    pltpu.make_async_copy(ew_h.at[pl.ds(off, EW)], ewm_v, sem).wait()

    for k in range(SLC // 16):
        slice_v[pl.ds(k * 16, 16)] = _zero_vec16()
    pltpu.sync_copy(slice_v, deg_s.at[pl.ds(s * SLC, SLC)])

    def mask_body(i, carry):
        sl = pl.ds(i * 16, 16)
        ewm_v[sl] = jnp.where(row_v[sl] == col_v[sl], 0.0, ewm_v[sl])
        return carry
    lax.fori_loop(0, EW // 16, mask_body, 0)
    _relayout_idx(row_v, row2_v, EW // C)

    plsc.subcore_barrier()

    def deg_body(j, carry):
        pltpu.sync_copy(ewm_v.at[pl.ds(j * C, C)], deg_s.at[row2_v.at[j]],
                        add=True)
        return carry
    lax.fori_loop(0, EW // C, deg_body, 0)

    plsc.subcore_barrier()

    pltpu.sync_copy(deg_s.at[pl.ds(s * SLC, SLC)],
                    deg_out.at[c, pl.ds(s * SLC, SLC)])
    pltpu.sync_copy(ewm_v, mew_out.at[pl.ds(off, EW)])


@functools.partial(
    pl.kernel,
    out_type=jax.ShapeDtypeStruct((NC, N, D), jnp.float32),
    mesh=_sc_mesh,
    scratch_types=[
        pltpu.VMEM((GW,), jnp.int32),
        pltpu.VMEM((GW,), jnp.int32),
        pltpu.VMEM((GRP, C), jnp.int32),
        pltpu.VMEM((GW,), jnp.float32),
        pltpu.VMEM((2, C, D), jnp.float32),
        pltpu.VMEM_SHARED((N, D), jnp.float32),
        pltpu.SemaphoreType.DMA((2,)),
        pltpu.SemaphoreType.DMA,
    ],
)
def _sc_prop(row_h, col_h, w_h, h_h, u_out,
             row_v, col_v, col2_v, w_v, gbuf, acc_s, sem_g, sem_st):
    c = lax.axis_index("c")
    s = lax.axis_index("s")
    off = (c * NS + s) * EW

    def zrow(i, carry):
        for q in range(D // 16):
            gbuf[0, i, pl.ds(q * 16, 16)] = _zero_vec16()
        return carry
    lax.fori_loop(0, C, zrow, 0)
    base = s * ZROWS
    for t in range(ZROWS // C):
        pltpu.sync_copy(gbuf.at[0], acc_s.at[pl.ds(base + t * C, C)])
    rem = ZROWS % C
    pltpu.sync_copy(gbuf.at[0, pl.ds(0, rem)],
                    acc_s.at[pl.ds(base + (ZROWS // C) * C, rem)])

    @pl.when(s == NS - 1)
    def _():
        pltpu.sync_copy(gbuf.at[0, pl.ds(0, TAIL)],
                        acc_s.at[pl.ds(NS * ZROWS, TAIL)])

    plsc.subcore_barrier()

    def _ridx(j):
        return row_v.at[pl.ds(j * C, C)]

    def group_body(g, carry):
        goff = off + g * GW
        pltpu.async_copy(row_h.at[pl.ds(goff, GW)], row_v, sem_st)
        pltpu.async_copy(col_h.at[pl.ds(goff, GW)], col_v, sem_st)
        pltpu.async_copy(w_h.at[pl.ds(goff, GW)], w_v, sem_st)
        pltpu.make_async_copy(row_h.at[pl.ds(goff, GW)], row_v,
                              sem_st).wait()
        pltpu.make_async_copy(col_h.at[pl.ds(goff, GW)], col_v,
                              sem_st).wait()
        pltpu.make_async_copy(w_h.at[pl.ds(goff, GW)], w_v, sem_st).wait()
        _relayout_idx(col_v, col2_v, GRP)
        pltpu.async_copy(h_h.at[_ridx(0)], gbuf.at[0], sem_g.at[0])

        def pair_body(i, carry2):
            j0 = 2 * i
            j1 = j0 + 1
            pltpu.make_async_copy(h_h.at[_ridx(j0)], gbuf.at[0],
                                  sem_g.at[0]).wait()
            pltpu.async_copy(h_h.at[_ridx(j1)], gbuf.at[1], sem_g.at[1])
            _scale_rows(gbuf, 0, w_v, j0)
            pltpu.sync_copy(gbuf.at[0], acc_s.at[col2_v.at[j0]], add=True)
            pltpu.async_copy(h_h.at[_ridx(j0 + 2)], gbuf.at[0],
                             sem_g.at[0])
            pltpu.make_async_copy(h_h.at[_ridx(j1)], gbuf.at[1],
                                  sem_g.at[1]).wait()
            _scale_rows(gbuf, 1, w_v, j1)
            pltpu.sync_copy(gbuf.at[1], acc_s.at[col2_v.at[j1]], add=True)
            return carry2
        lax.fori_loop(0, (GRP - 1) // 2, pair_body, carry)
        pltpu.make_async_copy(h_h.at[_ridx(GRP - 1)], gbuf.at[0],
                              sem_g.at[0]).wait()
        _scale_rows(gbuf, 0, w_v, GRP - 1)
        pltpu.sync_copy(gbuf.at[0], acc_s.at[col2_v.at[GRP - 1]], add=True)
        return carry
    lax.fori_loop(0, NG, group_body, 0)

    plsc.subcore_barrier()

    pltpu.sync_copy(acc_s.at[pl.ds(base, ZROWS)],
                    u_out.at[c, pl.ds(base, ZROWS)])

    @pl.when(s == NS - 1)
    def _():
        pltpu.sync_copy(acc_s.at[pl.ds(NS * ZROWS, TAIL)],
                        u_out.at[c, pl.ds(NS * ZROWS, TAIL)])


_BLK = 2000


def _tc_pre_body(x, d0, d1, w0, wlt, bl, y_o, dinv_o, xw0_o, xl_o):
    deg = d0[...] + d1[...]
    dinv = jnp.where(deg > 0, lax.rsqrt(jnp.where(deg > 0, deg, 1.0)), 0.0)
    dinv_o[...] = dinv
    xb = x[...]
    y_o[...] = xb * dinv
    xw0_o[...] = jnp.dot(xb, w0[...], preferred_element_type=jnp.float32)
    xl_o[...] = jnp.dot(xb, wlt[...], preferred_element_type=jnp.float32) + bl[...]


def _tc_mid_body(u, dinv, xw0, w1, s1_o, y2_o):
    dv = dinv[...]
    tx1 = -(u[0] + u[1]) * dv
    s1_o[...] = xw0[...] + jnp.dot(tx1, w1[...],
                                   preferred_element_type=jnp.float32)
    y2_o[...] = tx1 * dv


def _tc_final_body(u, dinv, x, s1, xl, w2, bc, out_o):
    tx2 = -2.0 * (u[0] + u[1]) * dinv[...] - x[...]
    g = (s1[...] + jnp.dot(tx2, w2[...], preferred_element_type=jnp.float32)
         + bc[...])
    out_o[...] = jnp.where(g >= 0, g, 0.01 * g) + xl[...]


def _row_spec():
    return pl.BlockSpec((_BLK, D), lambda i: (i, 0))


def _col_spec():
    return pl.BlockSpec((_BLK, 1), lambda i: (i, 0))


def _u_spec():
    return pl.BlockSpec((NC, _BLK, D), lambda i: (0, i, 0))


def _w_spec():
    return pl.BlockSpec((D, D), lambda i: (0, 0))


def _b_spec():
    return pl.BlockSpec((1, D), lambda i: (0, 0))


def kernel(x, edge_index, edge_attr, W_cheb, b_cheb, W_lin, b_lin):
    row1 = edge_index[0]
    col1 = edge_index[1]

    degp, mew = _sc_degree(row1, col1, edge_attr)
    d0 = degp[0, :N].reshape(N, 1)
    d1 = degp[1, :N].reshape(N, 1)

    grid = (N // _BLK,)
    y, dinv, xw0, xl = pl.pallas_call(
        _tc_pre_body,
        grid=grid,
        in_specs=[_row_spec(), _col_spec(), _col_spec(),
                  _w_spec(), _w_spec(), _b_spec()],
        out_specs=[_row_spec(), _col_spec(), _row_spec(), _row_spec()],
        out_shape=[
            jax.ShapeDtypeStruct((N, D), jnp.float32),
            jax.ShapeDtypeStruct((N, 1), jnp.float32),
            jax.ShapeDtypeStruct((N, D), jnp.float32),
            jax.ShapeDtypeStruct((N, D), jnp.float32),
        ],
    )(x, d0, d1, W_cheb[0], W_lin.T, b_lin.reshape(1, D))

    u1 = _sc_prop(row1, col1, mew, y)

    s1, y2 = pl.pallas_call(
        _tc_mid_body,
        grid=grid,
        in_specs=[_u_spec(), _col_spec(), _row_spec(), _w_spec()],
        out_specs=[_row_spec(), _row_spec()],
        out_shape=[jax.ShapeDtypeStruct((N, D), jnp.float32)] * 2,
    )(u1, dinv, xw0, W_cheb[1])

    u2 = _sc_prop(row1, col1, mew, y2)

    out = pl.pallas_call(
        _tc_final_body,
        grid=grid,
        in_specs=[_u_spec(), _col_spec(), _row_spec(),
                  _row_spec(), _row_spec(), _w_spec(), _b_spec()],
        out_specs=_row_spec(),
        out_shape=jax.ShapeDtypeStruct((N, D), jnp.float32),
    )(u2, dinv, x, s1, xl, W_cheb[2], b_cheb.reshape(1, D))
    return out


# speedup vs baseline: 1.2763x; 1.0192x over previous
import functools

import jax
import jax.numpy as jnp
from jax import lax
from jax.experimental import pallas as pl
from jax.experimental.pallas import tpu as pltpu
from jax.experimental.pallas import tpu_sc as plsc

N = 10000
E = 320000
D = 128
NC = 2
NS = 16
C = 80
NG = 5
GRP = 25
EW = E // (NC * NS)
GW = GRP * C
NPAD = 10240
SLC = NPAD // NS
ZROWS = 624
TAIL = N - NS * ZROWS


def _zero_vec16():
    return jnp.zeros((16,), jnp.float32)


def _scale_rows(gbuf, b, w_ref, j, g_lo=0, g_hi=C // 16):
    def grp(g, carry):
        wvec = w_ref[pl.ds(j * C + g * 16, 16)]
        for l in range(16):
            ws = wvec[l]
            e2 = g * 16 + l
            for q in range(D // 16):
                sl = pl.ds(q * 16, 16)
                gbuf[b, e2, sl] = gbuf[b, e2, sl] * ws
        return carry
    lax.fori_loop(g_lo, g_hi, grp, 0)


def _relayout_idx(src1d, dst2d, nrows):
    def body(j, carry):
        for k in range(C // 16):
            dst2d[j, pl.ds(k * 16, 16)] = src1d[pl.ds(j * C + k * 16, 16)]
        return carry
    lax.fori_loop(0, nrows, body, 0)


CA = 48
CB = C - CA


def _relayout_split(src1d, dst_a, dst_b, nrows):
    def body(j, carry):
        for k in range(CA // 16):
            dst_a[j, pl.ds(k * 16, 16)] = src1d[pl.ds(j * C + k * 16, 16)]
        for k in range(CB // 16):
            dst_b[j, pl.ds(k * 16, 16)] = src1d[pl.ds(j * C + CA + k * 16,
                                                      16)]
        return carry
    lax.fori_loop(0, nrows, body, 0)


_sc_mesh = plsc.VectorSubcoreMesh(core_axis_name="c", subcore_axis_name="s")


@functools.partial(
    pl.kernel,
    out_type=[
        jax.ShapeDtypeStruct((NC, NPAD), jnp.float32),
        jax.ShapeDtypeStruct((E,), jnp.float32),
    ],
    mesh=_sc_mesh,
    scratch_types=[
        pltpu.VMEM((EW,), jnp.int32),
        pltpu.VMEM((EW,), jnp.int32),
        pltpu.VMEM((EW,), jnp.float32),
        pltpu.VMEM((EW // C, C), jnp.int32),
        pltpu.VMEM((SLC,), jnp.float32),
        pltpu.VMEM_SHARED((NPAD,), jnp.float32),
        pltpu.SemaphoreType.DMA,
    ],
)
def _sc_degree(row_h, col_h, ew_h, deg_out, mew_out,
               row_v, col_v, ewm_v, row2_v, slice_v, deg_s, sem):
    c = lax.axis_index("c")
    s = lax.axis_index("s")
    off = (c * NS + s) * EW

    pltpu.async_copy(row_h.at[pl.ds(off, EW)], row_v, sem)
    pltpu.async_copy(col_h.at[pl.ds(off, EW)], col_v, sem)
    pltpu.async_copy(ew_h.at[pl.ds(off, EW)], ewm_v, sem)
    pltpu.make_async_copy(row_h.at[pl.ds(off, EW)], row_v, sem).wait()
    pltpu.make_async_copy(col_h.at[pl.ds(off, EW)], col_v, sem).wait()
    pltpu.make_async_copy(ew_h.at[pl.ds(off, EW)], ewm_v, sem).wait()

    for k in range(SLC // 16):
        slice_v[pl.ds(k * 16, 16)] = _zero_vec16()
    pltpu.sync_copy(slice_v, deg_s.at[pl.ds(s * SLC, SLC)])

    def mask_body(i, carry):
        sl = pl.ds(i * 16, 16)
        ewm_v[sl] = jnp.where(row_v[sl] == col_v[sl], 0.0, ewm_v[sl])
        return carry
    lax.fori_loop(0, EW // 16, mask_body, 0)
    _relayout_idx(row_v, row2_v, EW // C)

    plsc.subcore_barrier()

    def deg_body(j, carry):
        pltpu.sync_copy(ewm_v.at[pl.ds(j * C, C)], deg_s.at[row2_v.at[j]],
                        add=True)
        return carry
    lax.fori_loop(0, EW // C, deg_body, 0)

    plsc.subcore_barrier()

    pltpu.sync_copy(deg_s.at[pl.ds(s * SLC, SLC)],
                    deg_out.at[c, pl.ds(s * SLC, SLC)])
    pltpu.sync_copy(ewm_v, mew_out.at[pl.ds(off, EW)])


@functools.partial(
    pl.kernel,
    out_type=jax.ShapeDtypeStruct((NC, N, D), jnp.float32),
    mesh=_sc_mesh,
    scratch_types=[
        pltpu.VMEM((GW,), jnp.int32),
        pltpu.VMEM((GW,), jnp.int32),
        pltpu.VMEM((GRP, CA), jnp.int32),
        pltpu.VMEM((GRP, CB), jnp.int32),
        pltpu.VMEM((GW,), jnp.float32),
        pltpu.VMEM((2, C, D), jnp.float32),
        pltpu.VMEM_SHARED((N, D), jnp.float32),
        pltpu.SemaphoreType.DMA((2,)),
        pltpu.SemaphoreType.DMA,
        pltpu.SemaphoreType.DMA,
    ],
)
def _sc_prop(row_h, col_h, w_h, h_h, u_out,
             row_v, col_v, col2a_v, col2b_v, w_v, gbuf, acc_s,
             sem_g, sem_st, sem_s):
    c = lax.axis_index("c")
    s = lax.axis_index("s")
    off = (c * NS + s) * EW

    def zrow(i, carry):
        for q in range(D // 16):
            gbuf[0, i, pl.ds(q * 16, 16)] = _zero_vec16()
        return carry
    lax.fori_loop(0, C, zrow, 0)
    base = s * ZROWS
    for t in range(ZROWS // C):
        pltpu.sync_copy(gbuf.at[0], acc_s.at[pl.ds(base + t * C, C)])
    rem = ZROWS % C
    pltpu.sync_copy(gbuf.at[0, pl.ds(0, rem)],
                    acc_s.at[pl.ds(base + (ZROWS // C) * C, rem)])

    @pl.when(s == NS - 1)
    def _():
        pltpu.sync_copy(gbuf.at[0, pl.ds(0, TAIL)],
                        acc_s.at[pl.ds(NS * ZROWS, TAIL)])

    plsc.subcore_barrier()

    def _ridx(j):
        return row_v.at[pl.ds(j * C, C)]

    def group_body(g, carry):
        goff = off + g * GW
        pltpu.async_copy(row_h.at[pl.ds(goff, GW)], row_v, sem_st)
        pltpu.async_copy(col_h.at[pl.ds(goff, GW)], col_v, sem_st)
        pltpu.async_copy(w_h.at[pl.ds(goff, GW)], w_v, sem_st)
        pltpu.make_async_copy(row_h.at[pl.ds(goff, GW)], row_v,
                              sem_st).wait()
        pltpu.make_async_copy(col_h.at[pl.ds(goff, GW)], col_v,
                              sem_st).wait()
        pltpu.make_async_copy(w_h.at[pl.ds(goff, GW)], w_v, sem_st).wait()
        _relayout_split(col_v, col2a_v, col2b_v, GRP)
        pltpu.async_copy(h_h.at[_ridx(0)], gbuf.at[0], sem_g.at[0])

        def _scatter_chunk(b, j):
            _scale_rows(gbuf, b, w_v, j, 0, CA // 16)
            pltpu.async_copy(gbuf.at[b, pl.ds(0, CA)],
                             acc_s.at[col2a_v.at[j]], sem_s, add=True)
            _scale_rows(gbuf, b, w_v, j, CA // 16, C // 16)
            pltpu.sync_copy(gbuf.at[b, pl.ds(CA, CB)],
                            acc_s.at[col2b_v.at[j]], add=True)
            pltpu.make_async_copy(gbuf.at[b, pl.ds(0, CA)],
                                  acc_s.at[col2a_v.at[j]], sem_s).wait()

        def pair_body(i, carry2):
            j0 = 2 * i
            j1 = j0 + 1
            pltpu.make_async_copy(h_h.at[_ridx(j0)], gbuf.at[0],
                                  sem_g.at[0]).wait()
            pltpu.async_copy(h_h.at[_ridx(j1)], gbuf.at[1], sem_g.at[1])
            _scatter_chunk(0, j0)
            pltpu.async_copy(h_h.at[_ridx(j0 + 2)], gbuf.at[0],
                             sem_g.at[0])
            pltpu.make_async_copy(h_h.at[_ridx(j1)], gbuf.at[1],
                                  sem_g.at[1]).wait()
            _scatter_chunk(1, j1)
            return carry2
        lax.fori_loop(0, (GRP - 1) // 2, pair_body, carry)
        pltpu.make_async_copy(h_h.at[_ridx(GRP - 1)], gbuf.at[0],
                              sem_g.at[0]).wait()
        _scatter_chunk(0, GRP - 1)
        return carry
    lax.fori_loop(0, NG, group_body, 0)

    plsc.subcore_barrier()

    pltpu.sync_copy(acc_s.at[pl.ds(base, ZROWS)],
                    u_out.at[c, pl.ds(base, ZROWS)])

    @pl.when(s == NS - 1)
    def _():
        pltpu.sync_copy(acc_s.at[pl.ds(NS * ZROWS, TAIL)],
                        u_out.at[c, pl.ds(NS * ZROWS, TAIL)])


_BLK = 2000


def _tc_pre_body(x, d0, d1, w0, wlt, bl, y_o, dinv_o, xw0_o, xl_o):
    deg = d0[...] + d1[...]
    dinv = jnp.where(deg > 0, lax.rsqrt(jnp.where(deg > 0, deg, 1.0)), 0.0)
    dinv_o[...] = dinv
    xb = x[...]
    y_o[...] = xb * dinv
    xw0_o[...] = jnp.dot(xb, w0[...], preferred_element_type=jnp.float32)
    xl_o[...] = jnp.dot(xb, wlt[...], preferred_element_type=jnp.float32) + bl[...]


def _tc_mid_body(u, dinv, xw0, w1, s1_o, y2_o):
    dv = dinv[...]
    tx1 = -(u[0] + u[1]) * dv
    s1_o[...] = xw0[...] + jnp.dot(tx1, w1[...],
                                   preferred_element_type=jnp.float32)
    y2_o[...] = tx1 * dv


def _tc_final_body(u, dinv, x, s1, xl, w2, bc, out_o):
    tx2 = -2.0 * (u[0] + u[1]) * dinv[...] - x[...]
    g = (s1[...] + jnp.dot(tx2, w2[...], preferred_element_type=jnp.float32)
         + bc[...])
    out_o[...] = jnp.where(g >= 0, g, 0.01 * g) + xl[...]


def _row_spec():
    return pl.BlockSpec((_BLK, D), lambda i: (i, 0))


def _col_spec():
    return pl.BlockSpec((_BLK, 1), lambda i: (i, 0))


def _u_spec():
    return pl.BlockSpec((NC, _BLK, D), lambda i: (0, i, 0))


def _w_spec():
    return pl.BlockSpec((D, D), lambda i: (0, 0))


def _b_spec():
    return pl.BlockSpec((1, D), lambda i: (0, 0))


def kernel(x, edge_index, edge_attr, W_cheb, b_cheb, W_lin, b_lin):
    row1 = edge_index[0]
    col1 = edge_index[1]

    degp, mew = _sc_degree(row1, col1, edge_attr)
    d0 = degp[0, :N].reshape(N, 1)
    d1 = degp[1, :N].reshape(N, 1)

    grid = (N // _BLK,)
    y, dinv, xw0, xl = pl.pallas_call(
        _tc_pre_body,
        grid=grid,
        in_specs=[_row_spec(), _col_spec(), _col_spec(),
                  _w_spec(), _w_spec(), _b_spec()],
        out_specs=[_row_spec(), _col_spec(), _row_spec(), _row_spec()],
        out_shape=[
            jax.ShapeDtypeStruct((N, D), jnp.float32),
            jax.ShapeDtypeStruct((N, 1), jnp.float32),
            jax.ShapeDtypeStruct((N, D), jnp.float32),
            jax.ShapeDtypeStruct((N, D), jnp.float32),
        ],
    )(x, d0, d1, W_cheb[0], W_lin.T, b_lin.reshape(1, D))

    u1 = _sc_prop(row1, col1, mew, y)

    s1, y2 = pl.pallas_call(
        _tc_mid_body,
        grid=grid,
        in_specs=[_u_spec(), _col_spec(), _row_spec(), _w_spec()],
        out_specs=[_row_spec(), _row_spec()],
        out_shape=[jax.ShapeDtypeStruct((N, D), jnp.float32)] * 2,
    )(u1, dinv, xw0, W_cheb[1])

    u2 = _sc_prop(row1, col1, mew, y2)

    out = pl.pallas_call(
        _tc_final_body,
        grid=grid,
        in_specs=[_u_spec(), _col_spec(), _row_spec(),
                  _row_spec(), _row_spec(), _w_spec(), _b_spec()],
        out_specs=_row_spec(),
        out_shape=jax.ShapeDtypeStruct((N, D), jnp.float32),
    )(u2, dinv, x, s1, xl, W_cheb[2], b_cheb.reshape(1, D))
    return out


# speedup vs baseline: 1.3185x; 1.0330x over previous
import functools

import jax
import jax.numpy as jnp
from jax import lax
from jax.experimental import pallas as pl
from jax.experimental.pallas import tpu as pltpu
from jax.experimental.pallas import tpu_sc as plsc

N = 10000
E = 320000
D = 128
NC = 2
NS = 16
C = 80
NG = 5
GRP = 25
EW = E // (NC * NS)
GW = GRP * C
AW = 10112
AG = 2176
NPAD = 10240
SLC = NPAD // NS
ZROWS = 624
TAIL = N - NS * ZROWS


def _zero_vec16():
    return jnp.zeros((16,), jnp.float32)


def _scale_rows(gbuf, b, w_ref, j, g_lo=0, g_hi=C // 16):
    def grp(g, carry):
        wvec = w_ref[pl.ds(j * C + g * 16, 16)]
        for l in range(16):
            ws = wvec[l]
            e2 = g * 16 + l
            for q in range(D // 16):
                sl = pl.ds(q * 16, 16)
                gbuf[b, e2, sl] = gbuf[b, e2, sl] * ws
        return carry
    lax.fori_loop(g_lo, g_hi, grp, 0)


def _relayout_idx(src1d, base, dst2d, nrows):
    def body(j, carry):
        for k in range(C // 16):
            dst2d[j, pl.ds(k * 16, 16)] = src1d[pl.ds(base + j * C + k * 16,
                                                      16)]
        return carry
    lax.fori_loop(0, nrows, body, 0)


CA = 48
CB = C - CA


def _relayout_split(src1d, base, dst_a, dst_b, nrows):
    def body(j, carry):
        for k in range(CA // 16):
            dst_a[j, pl.ds(k * 16, 16)] = src1d[pl.ds(base + j * C + k * 16,
                                                      16)]
        for k in range(CB // 16):
            dst_b[j, pl.ds(k * 16, 16)] = src1d[
                pl.ds(base + j * C + CA + k * 16, 16)]
        return carry
    lax.fori_loop(0, nrows, body, 0)


_sc_mesh = plsc.VectorSubcoreMesh(core_axis_name="c", subcore_axis_name="s")


@functools.partial(
    pl.kernel,
    out_type=[
        jax.ShapeDtypeStruct((NC, NPAD), jnp.float32),
        jax.ShapeDtypeStruct((E,), jnp.float32),
    ],
    mesh=_sc_mesh,
    scratch_types=[
        pltpu.VMEM((AW,), jnp.int32),
        pltpu.VMEM((AW,), jnp.int32),
        pltpu.VMEM((EW,), jnp.float32),
        pltpu.VMEM((EW // C, C), jnp.int32),
        pltpu.VMEM((SLC,), jnp.float32),
        pltpu.VMEM_SHARED((NPAD,), jnp.float32),
        pltpu.SemaphoreType.DMA,
    ],
)
def _sc_degree(ei_h, ew_h, deg_out, mew_out,
               row_v, col_v, ewm_v, row2_v, slice_v, deg_s, sem):
    c = lax.axis_index("c")
    s = lax.axis_index("s")
    off = (c * NS + s) * EW
    aoff = (off // 128) * 128
    r = off - aoff

    pltpu.async_copy(ei_h.at[0, pl.ds(aoff, AW)], row_v, sem)
    pltpu.async_copy(ei_h.at[1, pl.ds(aoff, AW)], col_v, sem)
    pltpu.async_copy(ew_h.at[pl.ds(off, EW)], ewm_v, sem)
    pltpu.make_async_copy(ei_h.at[0, pl.ds(aoff, AW)], row_v, sem).wait()
    pltpu.make_async_copy(ei_h.at[1, pl.ds(aoff, AW)], col_v, sem).wait()
    pltpu.make_async_copy(ew_h.at[pl.ds(off, EW)], ewm_v, sem).wait()

    for k in range(SLC // 16):
        slice_v[pl.ds(k * 16, 16)] = _zero_vec16()
    pltpu.sync_copy(slice_v, deg_s.at[pl.ds(s * SLC, SLC)])

    def mask_body(i, carry):
        sl = pl.ds(i * 16, 16)
        sr = pl.ds(r + i * 16, 16)
        ewm_v[sl] = jnp.where(row_v[sr] == col_v[sr], 0.0, ewm_v[sl])
        return carry
    lax.fori_loop(0, EW // 16, mask_body, 0)
    _relayout_idx(row_v, r, row2_v, EW // C)

    plsc.subcore_barrier()

    def deg_body(j, carry):
        pltpu.sync_copy(ewm_v.at[pl.ds(j * C, C)], deg_s.at[row2_v.at[j]],
                        add=True)
        return carry
    lax.fori_loop(0, EW // C, deg_body, 0)

    plsc.subcore_barrier()

    pltpu.sync_copy(deg_s.at[pl.ds(s * SLC, SLC)],
                    deg_out.at[c, pl.ds(s * SLC, SLC)])
    pltpu.sync_copy(ewm_v, mew_out.at[pl.ds(off, EW)])


@functools.partial(
    pl.kernel,
    out_type=jax.ShapeDtypeStruct((NC, N, D), jnp.float32),
    mesh=_sc_mesh,
    scratch_types=[
        pltpu.VMEM((AG,), jnp.int32),
        pltpu.VMEM((AG,), jnp.int32),
        pltpu.VMEM((GRP, CA), jnp.int32),
        pltpu.VMEM((GRP, CB), jnp.int32),
        pltpu.VMEM((GW,), jnp.float32),
        pltpu.VMEM((2, C, D), jnp.float32),
        pltpu.VMEM_SHARED((N, D), jnp.float32),
        pltpu.SemaphoreType.DMA((2,)),
        pltpu.SemaphoreType.DMA,
        pltpu.SemaphoreType.DMA,
    ],
)
def _sc_prop(ei_h, w_h, h_h, u_out,
             row_v, col_v, col2a_v, col2b_v, w_v, gbuf, acc_s,
             sem_g, sem_st, sem_s):
    c = lax.axis_index("c")
    s = lax.axis_index("s")
    off = (c * NS + s) * EW

    def zrow(i, carry):
        for q in range(D // 16):
            gbuf[0, i, pl.ds(q * 16, 16)] = _zero_vec16()
        return carry
    lax.fori_loop(0, C, zrow, 0)
    base = s * ZROWS
    for t in range(ZROWS // C):
        pltpu.sync_copy(gbuf.at[0], acc_s.at[pl.ds(base + t * C, C)])
    rem = ZROWS % C
    pltpu.sync_copy(gbuf.at[0, pl.ds(0, rem)],
                    acc_s.at[pl.ds(base + (ZROWS // C) * C, rem)])

    @pl.when(s == NS - 1)
    def _():
        pltpu.sync_copy(gbuf.at[0, pl.ds(0, TAIL)],
                        acc_s.at[pl.ds(NS * ZROWS, TAIL)])

    plsc.subcore_barrier()

    def group_body(g, carry):
        goff = off + g * GW
        aoff = jnp.minimum((goff // 128) * 128, E - AG)
        rg = goff - aoff

        def _ridx(j):
            return row_v.at[pl.ds(rg + j * C, C)]

        pltpu.async_copy(ei_h.at[0, pl.ds(aoff, AG)], row_v, sem_st)
        pltpu.async_copy(ei_h.at[1, pl.ds(aoff, AG)], col_v, sem_st)
        pltpu.async_copy(w_h.at[pl.ds(goff, GW)], w_v, sem_st)
        pltpu.make_async_copy(ei_h.at[0, pl.ds(aoff, AG)], row_v,
                              sem_st).wait()
        pltpu.make_async_copy(ei_h.at[1, pl.ds(aoff, AG)], col_v,
                              sem_st).wait()
        pltpu.make_async_copy(w_h.at[pl.ds(goff, GW)], w_v, sem_st).wait()
        _relayout_split(col_v, rg, col2a_v, col2b_v, GRP)
        pltpu.async_copy(h_h.at[_ridx(0)], gbuf.at[0], sem_g.at[0])

        def _scatter_chunk(b, j):
            _scale_rows(gbuf, b, w_v, j, 0, CA // 16)
            pltpu.async_copy(gbuf.at[b, pl.ds(0, CA)],
                             acc_s.at[col2a_v.at[j]], sem_s, add=True)
            _scale_rows(gbuf, b, w_v, j, CA // 16, C // 16)
            pltpu.sync_copy(gbuf.at[b, pl.ds(CA, CB)],
                            acc_s.at[col2b_v.at[j]], add=True)
            pltpu.make_async_copy(gbuf.at[b, pl.ds(0, CA)],
                                  acc_s.at[col2a_v.at[j]], sem_s).wait()

        def pair_body(i, carry2):
            j0 = 2 * i
            j1 = j0 + 1
            pltpu.make_async_copy(h_h.at[_ridx(j0)], gbuf.at[0],
                                  sem_g.at[0]).wait()
            pltpu.async_copy(h_h.at[_ridx(j1)], gbuf.at[1], sem_g.at[1])
            _scatter_chunk(0, j0)
            pltpu.async_copy(h_h.at[_ridx(j0 + 2)], gbuf.at[0],
                             sem_g.at[0])
            pltpu.make_async_copy(h_h.at[_ridx(j1)], gbuf.at[1],
                                  sem_g.at[1]).wait()
            _scatter_chunk(1, j1)
            return carry2
        lax.fori_loop(0, (GRP - 1) // 2, pair_body, carry)
        pltpu.make_async_copy(h_h.at[_ridx(GRP - 1)], gbuf.at[0],
                              sem_g.at[0]).wait()
        _scatter_chunk(0, GRP - 1)
        return carry
    lax.fori_loop(0, NG, group_body, 0)

    plsc.subcore_barrier()

    pltpu.sync_copy(acc_s.at[pl.ds(base, ZROWS)],
                    u_out.at[c, pl.ds(base, ZROWS)])

    @pl.when(s == NS - 1)
    def _():
        pltpu.sync_copy(acc_s.at[pl.ds(NS * ZROWS, TAIL)],
                        u_out.at[c, pl.ds(NS * ZROWS, TAIL)])


_BLK = 2000


def _tc_pre_body(x, d0, d1, w0, wlt, bl, y_o, dinv_o, xw0_o, xl_o):
    deg = d0[...] + d1[...]
    dinv = jnp.where(deg > 0, lax.rsqrt(jnp.where(deg > 0, deg, 1.0)), 0.0)
    dinv_o[...] = dinv
    xb = x[...]
    y_o[...] = xb * dinv
    xw0_o[...] = jnp.dot(xb, w0[...], preferred_element_type=jnp.float32)
    xl_o[...] = jnp.dot(xb, wlt[...], preferred_element_type=jnp.float32) + bl[...]


def _tc_mid_body(u, dinv, xw0, w1, s1_o, y2_o):
    dv = dinv[...]
    tx1 = -(u[0] + u[1]) * dv
    s1_o[...] = xw0[...] + jnp.dot(tx1, w1[...],
                                   preferred_element_type=jnp.float32)
    y2_o[...] = tx1 * dv


def _tc_final_body(u, dinv, x, s1, xl, w2, bc, out_o):
    tx2 = -2.0 * (u[0] + u[1]) * dinv[...] - x[...]
    g = (s1[...] + jnp.dot(tx2, w2[...], preferred_element_type=jnp.float32)
         + bc[...])
    out_o[...] = jnp.where(g >= 0, g, 0.01 * g) + xl[...]


def _row_spec():
    return pl.BlockSpec((_BLK, D), lambda i: (i, 0))


def _col_spec():
    return pl.BlockSpec((_BLK, 1), lambda i: (i, 0))


def _u_spec():
    return pl.BlockSpec((NC, _BLK, D), lambda i: (0, i, 0))


def _w_spec():
    return pl.BlockSpec((D, D), lambda i: (0, 0))


def _b_spec():
    return pl.BlockSpec((1, D), lambda i: (0, 0))


def kernel(x, edge_index, edge_attr, W_cheb, b_cheb, W_lin, b_lin):
    degp, mew = _sc_degree(edge_index, edge_attr)
    d0 = degp[0, :N].reshape(N, 1)
    d1 = degp[1, :N].reshape(N, 1)

    grid = (N // _BLK,)
    y, dinv, xw0, xl = pl.pallas_call(
        _tc_pre_body,
        grid=grid,
        in_specs=[_row_spec(), _col_spec(), _col_spec(),
                  _w_spec(), _w_spec(), _b_spec()],
        out_specs=[_row_spec(), _col_spec(), _row_spec(), _row_spec()],
        out_shape=[
            jax.ShapeDtypeStruct((N, D), jnp.float32),
            jax.ShapeDtypeStruct((N, 1), jnp.float32),
            jax.ShapeDtypeStruct((N, D), jnp.float32),
            jax.ShapeDtypeStruct((N, D), jnp.float32),
        ],
    )(x, d0, d1, W_cheb[0], W_lin.T, b_lin.reshape(1, D))

    u1 = _sc_prop(edge_index, mew, y)

    s1, y2 = pl.pallas_call(
        _tc_mid_body,
        grid=grid,
        in_specs=[_u_spec(), _col_spec(), _row_spec(), _w_spec()],
        out_specs=[_row_spec(), _row_spec()],
        out_shape=[jax.ShapeDtypeStruct((N, D), jnp.float32)] * 2,
    )(u1, dinv, xw0, W_cheb[1])

    u2 = _sc_prop(edge_index, mew, y2)

    out = pl.pallas_call(
        _tc_final_body,
        grid=grid,
        in_specs=[_u_spec(), _col_spec(), _row_spec(),
                  _row_spec(), _row_spec(), _w_spec(), _b_spec()],
        out_specs=_row_spec(),
        out_shape=jax.ShapeDtypeStruct((N, D), jnp.float32),
    )(u2, dinv, x, s1, xl, W_cheb[2], b_cheb.reshape(1, D))
    return out
